# baseline (speedup 1.0000x reference)
import functools

import jax
import jax.numpy as jnp
from jax import lax
from jax.experimental import pallas as pl
from jax.experimental.pallas import tpu as pltpu

N_NODE = 2048
N_EDGE = 32768
E_CHUNK = 1024
NODES_PER_BLOCK = 64
N_BLOCKS = N_NODE // NODES_PER_BLOCK
ROWS_PER_BLOCK = 3 * NODES_PER_BLOCK


def _edge_blocks_kernel(src_ref, dst_ref, em_ref, a_ref, i_ref, coords_ref,
                        out_ref):
    src = src_ref[...]
    dst = dst_ref[...]
    coords = coords_ref[...]
    iota = lax.broadcasted_iota(jnp.int32, (E_CHUNK, N_NODE), 1)
    oh_s = (src == iota).astype(jnp.float32)
    oh_d = (dst == iota).astype(jnp.float32)
    xs = jnp.dot(oh_s, coords, preferred_element_type=jnp.float32,
                 precision=lax.Precision.HIGHEST)
    xd = jnp.dot(oh_d, coords, preferred_element_type=jnp.float32,
                 precision=lax.Precision.HIGHEST)
    dx = xs[:, 0:1] - xd[:, 0:1]
    dy = xs[:, 1:2] - xd[:, 1:2]
    L = jnp.sqrt(dx * dx + dy * dy)
    em = em_ref[...]
    krot = em * i_ref[...] / (L * L * L)
    klin = em * a_ref[...] / L
    cos = dx / L
    sin = -dy / L
    ss = sin * sin
    cc = cos * cos
    sc = sin * cos
    Ls = 6.0 * L * sin
    Lc = 6.0 * L * cos
    L2 = 2.0 * L * L
    L4 = 4.0 * L * L
    z = jnp.zeros_like(L)
    rot = [
        12 * ss, 12 * sc, -Ls, -12 * ss, -12 * sc, -Ls,
        12 * sc, 12 * cc, -Lc, -12 * sc, -12 * cc, -Lc,
        -Ls, -Lc, L4, Ls, Lc, L2,
        -12 * ss, -12 * sc, Ls, 12 * ss, 12 * sc, Ls,
        -12 * sc, -12 * cc, Lc, 12 * sc, 12 * cc, Lc,
        -Ls, -Lc, L2, Ls, Lc, L4,
    ]
    lin = [
        cc, -sc, z, -cc, sc, z,
        -sc, ss, z, sc, -ss, z,
        z, z, z, z, z, z,
        -cc, sc, z, cc, -sc, z,
        sc, -ss, z, -sc, ss, z,
        z, z, z, z, z, z,
    ]
    cols = [r * krot + l * klin for r, l in zip(rot, lin)]
    out_ref[...] = jnp.concatenate(cols, axis=1)


PAD_COLS = 3 * N_NODE + 256
ZERO_SLOT = 2 * N_EDGE


def _assemble_kernel(perm_ref, rn_ref, sd_ref, st_ref, vals_ref,
                     acc_ref):
    b = pl.program_id(0)
    acc_ref[...] = jnp.zeros((8 * NODES_PER_BLOCK, PAD_COLS), jnp.float32)
    base_node = b * NODES_PER_BLOCK
    lane = lax.broadcasted_iota(jnp.int32, (8, 256), 1)

    lo = st_ref[b]
    hi = st_ref[b + 1]

    def body(g, _):
      for j in range(8):
        r = g * 8 + j
        valid = jnp.logical_and(r >= lo, r < hi)
        p = jnp.where(valid, perm_ref[r], ZERO_SLOT)
        r0 = (jnp.where(valid, rn_ref[r], base_node) - base_node) * 8
        sd = jnp.where(valid, sd_ref[r], 0)
        cs = (sd // N_NODE) * 3
        cd = (sd % N_NODE) * 3
        cell = vals_ref[pl.ds(8 * (p // 16), 8), :]
        tile = pltpu.roll(cell, (128 - 8 * (p % 16)) % 128, axis=1)
        tile = jnp.concatenate([tile, jnp.zeros((8, 128), jnp.float32)],
                               axis=1)
        vs = jnp.where(lane < 3, tile, 0.0)
        vd = jnp.where(lane < 3, pltpu.roll(tile, 253, axis=1), 0.0)

        for col, upd in ((cs, vs), (cd, vd)):
            cb = (col // 128) * 128
            win = (pl.ds(r0, 8), pl.ds(cb, 256))
            acc_ref[win] = acc_ref[win] + pltpu.roll(upd, col % 128, axis=1)
      return 0

    lax.fori_loop(lo // 8, (hi + 7) // 8, body, 0)


@jax.jit
def kernel(coordinates, delta, edge_src, edge_dst, E_mod, A, I):
    src = edge_src.astype(jnp.int32)
    dst = edge_dst.astype(jnp.int32)
    coords_upd = coordinates + delta

    kmat = pl.pallas_call(
        _edge_blocks_kernel,
        grid=(N_EDGE // E_CHUNK,),
        in_specs=[
            pl.BlockSpec((E_CHUNK, 1), lambda i: (i, 0)),
            pl.BlockSpec((E_CHUNK, 1), lambda i: (i, 0)),
            pl.BlockSpec((E_CHUNK, 1), lambda i: (i, 0)),
            pl.BlockSpec((E_CHUNK, 1), lambda i: (i, 0)),
            pl.BlockSpec((E_CHUNK, 1), lambda i: (i, 0)),
            pl.BlockSpec((N_NODE, 2), lambda i: (0, 0)),
        ],
        out_specs=pl.BlockSpec((E_CHUNK, 36), lambda i: (i, 0)),
        out_shape=jax.ShapeDtypeStruct((N_EDGE, 36), jnp.float32),
    )(src[:, None], dst[:, None], E_mod[:, None], A[:, None], I[:, None],
      coords_upd)

    k6 = kmat.reshape(N_EDGE, 6, 6)
    tiles = jnp.concatenate([k6[:, 0:3, :], k6[:, 3:6, :]], axis=0)
    ng = (2 * N_EDGE) // 16
    vp = tiles.reshape(ng, 16, 3, 6).transpose(0, 2, 1, 3)
    vp = jnp.pad(vp, ((0, 0), (0, 5), (0, 0), (0, 2)))
    vals_pad = vp.reshape(ng, 8, 128).reshape(ng * 8, 128)
    vals_pad = jnp.concatenate(
        [vals_pad, jnp.zeros((8, 128), jnp.float32)], axis=0)

    row_node = jnp.concatenate([src, dst])
    sd_packed = jnp.concatenate([src * N_NODE + dst] * 2)
    perm = jnp.argsort(row_node).astype(jnp.int32)
    rn_sorted = row_node[perm]
    sd_sorted = sd_packed[perm]
    bounds = jnp.arange(N_BLOCKS + 1, dtype=jnp.int32) * NODES_PER_BLOCK
    start = jnp.searchsorted(rn_sorted, bounds).astype(jnp.int32)

    full = pl.pallas_call(
        _assemble_kernel,
        grid_spec=pltpu.PrefetchScalarGridSpec(
            num_scalar_prefetch=4,
            grid=(N_BLOCKS,),
            in_specs=[
                pl.BlockSpec((2 * N_EDGE // 16 * 8 + 8, 128), lambda b, *_: (0, 0)),
            ],
            out_specs=pl.BlockSpec((8 * NODES_PER_BLOCK, PAD_COLS),
                                   lambda b, *_: (b, 0)),
        ),
        out_shape=jax.ShapeDtypeStruct((8 * N_NODE, PAD_COLS), jnp.float32),
    )(perm, rn_sorted, sd_sorted, start, vals_pad)
    return full.reshape(N_NODE, 8, PAD_COLS)[:, 0:3, :3 * N_NODE].reshape(
        3 * N_NODE, 3 * N_NODE)


# speedup vs baseline: 37.6588x; 37.6588x over previous
import functools

import jax
import jax.numpy as jnp
from jax import lax
from jax.experimental import pallas as pl
from jax.experimental.pallas import tpu as pltpu

N_NODE = 2048
N_EDGE = 32768
E_CHUNK = 1024
NODES_PER_BLOCK = 64
N_BLOCKS = N_NODE // NODES_PER_BLOCK
ROWS_PER_BLOCK = 3 * NODES_PER_BLOCK


def _edge_blocks_kernel(src_ref, dst_ref, em_ref, a_ref, i_ref, coords_ref,
                        out_ref):
    src = src_ref[...]
    dst = dst_ref[...]
    coords = coords_ref[...]
    iota = lax.broadcasted_iota(jnp.int32, (E_CHUNK, N_NODE), 1)
    oh_s = (src == iota).astype(jnp.float32)
    oh_d = (dst == iota).astype(jnp.float32)
    xs = jnp.dot(oh_s, coords, preferred_element_type=jnp.float32,
                 precision=lax.Precision.HIGHEST)
    xd = jnp.dot(oh_d, coords, preferred_element_type=jnp.float32,
                 precision=lax.Precision.HIGHEST)
    dx = xs[:, 0:1] - xd[:, 0:1]
    dy = xs[:, 1:2] - xd[:, 1:2]
    L = jnp.sqrt(dx * dx + dy * dy)
    em = em_ref[...]
    krot = em * i_ref[...] / (L * L * L)
    klin = em * a_ref[...] / L
    cos = dx / L
    sin = -dy / L
    ss = sin * sin
    cc = cos * cos
    sc = sin * cos
    Ls = 6.0 * L * sin
    Lc = 6.0 * L * cos
    L2 = 2.0 * L * L
    L4 = 4.0 * L * L
    z = jnp.zeros_like(L)
    rot = [
        12 * ss, 12 * sc, -Ls, -12 * ss, -12 * sc, -Ls,
        12 * sc, 12 * cc, -Lc, -12 * sc, -12 * cc, -Lc,
        -Ls, -Lc, L4, Ls, Lc, L2,
        -12 * ss, -12 * sc, Ls, 12 * ss, 12 * sc, Ls,
        -12 * sc, -12 * cc, Lc, 12 * sc, 12 * cc, Lc,
        -Ls, -Lc, L2, Ls, Lc, L4,
    ]
    lin = [
        cc, -sc, z, -cc, sc, z,
        -sc, ss, z, sc, -ss, z,
        z, z, z, z, z, z,
        -cc, sc, z, cc, -sc, z,
        sc, -ss, z, -sc, ss, z,
        z, z, z, z, z, z,
    ]
    cols = [r * krot + l * klin for r, l in zip(rot, lin)]
    out_ref[...] = jnp.concatenate(cols, axis=1)


PAD_COLS = 3 * N_NODE + 256


def _assemble_kernel(perm_ref, rn_ref, sd_ref, st_ref, vals_ref,
                     acc_ref):
    b = pl.program_id(0)
    acc_ref[...] = jnp.zeros((8 * NODES_PER_BLOCK, PAD_COLS), jnp.float32)
    base_node = b * NODES_PER_BLOCK
    lane = lax.broadcasted_iota(jnp.int32, (8, 256), 1)

    def body(r, _):
        p = perm_ref[r]
        r0 = (rn_ref[r] - base_node) * 8
        sd = sd_ref[r]
        cs = (sd // N_NODE) * 3
        cd = (sd % N_NODE) * 3
        cell = vals_ref[pl.ds(8 * (p // 16), 8), :]
        tile = pltpu.roll(cell, (128 - 8 * (p % 16)) % 128, axis=1)
        tile = jnp.concatenate([tile, jnp.zeros((8, 128), jnp.float32)],
                               axis=1)
        vs = jnp.where(lane < 3, tile, 0.0)
        vd = jnp.where(lane < 3, pltpu.roll(tile, 253, axis=1), 0.0)

        for col, upd in ((cs, vs), (cd, vd)):
            cb = (col // 128) * 128
            win = (pl.ds(r0, 8), pl.ds(cb, 256))
            acc_ref[win] = acc_ref[win] + pltpu.roll(upd, col % 128, axis=1)
        return 0

    lax.fori_loop(st_ref[b], st_ref[b + 1], body, 0)


@jax.jit
def kernel(coordinates, delta, edge_src, edge_dst, E_mod, A, I):
    src = edge_src.astype(jnp.int32)
    dst = edge_dst.astype(jnp.int32)
    coords_upd = coordinates + delta

    kmat = pl.pallas_call(
        _edge_blocks_kernel,
        grid=(N_EDGE // E_CHUNK,),
        in_specs=[
            pl.BlockSpec((E_CHUNK, 1), lambda i: (i, 0)),
            pl.BlockSpec((E_CHUNK, 1), lambda i: (i, 0)),
            pl.BlockSpec((E_CHUNK, 1), lambda i: (i, 0)),
            pl.BlockSpec((E_CHUNK, 1), lambda i: (i, 0)),
            pl.BlockSpec((E_CHUNK, 1), lambda i: (i, 0)),
            pl.BlockSpec((N_NODE, 2), lambda i: (0, 0)),
        ],
        out_specs=pl.BlockSpec((E_CHUNK, 36), lambda i: (i, 0)),
        out_shape=jax.ShapeDtypeStruct((N_EDGE, 36), jnp.float32),
    )(src[:, None], dst[:, None], E_mod[:, None], A[:, None], I[:, None],
      coords_upd)

    k6 = kmat.reshape(N_EDGE, 6, 6)
    tiles = jnp.concatenate([k6[:, 0:3, :], k6[:, 3:6, :]], axis=0)
    ng = (2 * N_EDGE) // 16
    vp = tiles.reshape(ng, 16, 3, 6).transpose(0, 2, 1, 3)
    vp = jnp.pad(vp, ((0, 0), (0, 5), (0, 0), (0, 2)))
    vals_pad = vp.reshape(ng, 8, 128).reshape(ng * 8, 128)

    row_node = jnp.concatenate([src, dst])
    sd_packed = jnp.concatenate([src * N_NODE + dst] * 2)
    perm = jnp.argsort(row_node).astype(jnp.int32)
    rn_sorted = row_node[perm]
    sd_sorted = sd_packed[perm]
    bounds = jnp.arange(N_BLOCKS + 1, dtype=jnp.int32) * NODES_PER_BLOCK
    start = jnp.searchsorted(rn_sorted, bounds).astype(jnp.int32)

    full = pl.pallas_call(
        _assemble_kernel,
        grid_spec=pltpu.PrefetchScalarGridSpec(
            num_scalar_prefetch=4,
            grid=(N_BLOCKS,),
            in_specs=[
                pl.BlockSpec((2 * N_EDGE // 16 * 8, 128), lambda b, *_: (0, 0)),
            ],
            out_specs=pl.BlockSpec((8 * NODES_PER_BLOCK, PAD_COLS),
                                   lambda b, *_: (b, 0)),
        ),
        out_shape=jax.ShapeDtypeStruct((8 * N_NODE, PAD_COLS), jnp.float32),
    )(perm, rn_sorted, sd_sorted, start, vals_pad)
    return full.reshape(N_NODE, 8, PAD_COLS)[:, 0:3, :3 * N_NODE].reshape(
        3 * N_NODE, 3 * N_NODE)
